# Initial kernel scaffold; baseline (speedup 1.0000x reference)
import jax
import jax.numpy as jnp
from jax.experimental import pallas as pl


def kernel(x, absolute_pos_embedding, file_table, rank_table, diag_table, anti_diag_table):
    raise NotImplementedError("write your pallas kernel here")


# speedup vs baseline: 1.0232x; 1.0232x over previous
import functools

import jax
import jax.numpy as jnp
import numpy as np
from jax.experimental import pallas as pl

D_MODEL = 256
SEQ = 64
BATCH_BLOCK = 128


def _pos_add_kernel(x_ref, abs_ref, file_ref, rank_ref, diag_ref, anti_ref, o_ref):
    file_emb = jnp.tile(file_ref[...], (8, 1))
    rank_emb = jnp.repeat(rank_ref[...], 8, axis=0)
    row = jax.lax.broadcasted_iota(jnp.int32, (SEQ, 15), 0)
    col = jax.lax.broadcasted_iota(jnp.int32, (SEQ, 15), 1)
    diag_ids = row // 8 + row % 8
    anti_ids = row // 8 - row % 8 + 7
    diag_oh = (col == diag_ids).astype(jnp.float32)
    anti_oh = (col == anti_ids).astype(jnp.float32)
    diag_emb = jnp.dot(diag_oh, diag_ref[...], preferred_element_type=jnp.float32,
                       precision=jax.lax.Precision.HIGHEST)
    anti_emb = jnp.dot(anti_oh, anti_ref[...], preferred_element_type=jnp.float32,
                       precision=jax.lax.Precision.HIGHEST)
    pos = abs_ref[0] + file_emb + rank_emb + diag_emb + anti_emb
    o_ref[...] = x_ref[...] + pos[None, :, :]


@jax.jit
def kernel(x, absolute_pos_embedding, file_table, rank_table, diag_table, anti_diag_table):
    batch, seq, d = x.shape
    grid = (batch // BATCH_BLOCK,)
    return pl.pallas_call(
        _pos_add_kernel,
        grid=grid,
        in_specs=[
            pl.BlockSpec((BATCH_BLOCK, seq, d), lambda i: (i, 0, 0)),
            pl.BlockSpec((1, seq, d), lambda i: (0, 0, 0)),
            pl.BlockSpec((8, d), lambda i: (0, 0)),
            pl.BlockSpec((8, d), lambda i: (0, 0)),
            pl.BlockSpec((15, d), lambda i: (0, 0)),
            pl.BlockSpec((15, d), lambda i: (0, 0)),
        ],
        out_specs=pl.BlockSpec((BATCH_BLOCK, seq, d), lambda i: (i, 0, 0)),
        out_shape=jax.ShapeDtypeStruct(x.shape, x.dtype),
    )(x, absolute_pos_embedding, file_table, rank_table, diag_table, anti_diag_table)
